# Initial kernel scaffold; baseline (speedup 1.0000x reference)
import jax
import jax.numpy as jnp
from jax.experimental import pallas as pl


def kernel(boxes, scores):
    raise NotImplementedError("write your pallas kernel here")


# speedup vs baseline: 5.4359x; 5.4359x over previous
import functools

import jax
import jax.numpy as jnp
from jax import lax
from jax.experimental import pallas as pl
from jax.experimental.pallas import tpu as pltpu
from jax.experimental.pallas import tpu_sc as plsc

N = 20000
NP = 20480
NW = 16
PW = NP // NW
NCH = PW // 16
MAX_DET = 100
SCORE_THRESH = 0.05
NMS_THRESH = 0.5
NEG_INF = float("-inf")


def _nms_body(cat_hbm, sc_hbm, out_hbm, cat_v, work_v, gio_v, stage_v,
              allc_v, out_v, cands_sh):
    wid = lax.axis_index("s")
    base = wid * PW
    base_f = base.astype(jnp.float32)
    iota = lax.broadcasted_iota(jnp.int32, (16,), 0)
    iota_f = iota.astype(jnp.float32)
    lane5 = iota * 16 + 5
    gidx_plane = jnp.minimum(iota, 4) * PW

    for k in range(4):
        pltpu.sync_copy(cat_hbm.at[pl.ds(k * NP + base, PW)],
                        cat_v.at[pl.ds(k * PW, PW)])
    pltpu.sync_copy(sc_hbm.at[pl.ds(base, PW)], work_v)

    def init_body(c, _):
        ds = pl.ds(c * 16, 16)
        x1 = cat_v[pl.ds(c * 16, 16)]
        y1 = cat_v[pl.ds(PW + c * 16, 16)]
        x2 = cat_v[pl.ds(2 * PW + c * 16, 16)]
        y2 = cat_v[pl.ds(3 * PW + c * 16, 16)]
        area = jnp.maximum(x2 - x1, 0.0) * jnp.maximum(y2 - y1, 0.0)
        cat_v[pl.ds(4 * PW + c * 16, 16)] = area
        s = work_v[ds]
        work_v[ds] = jnp.where(s >= SCORE_THRESH, s, NEG_INF)
        gio_v[ds] = base_f + c * 16.0 + iota_f
        return 0
    lax.fori_loop(0, NCH, init_body, 0, unroll=False)

    def round_body(it, _):
        def amax_body(c, carry):
            m, mi = carry
            v = work_v[pl.ds(c * 16, 16)]
            gi = gio_v[pl.ds(c * 16, 16)]
            upd = v > m
            return jnp.where(upd, v, m), jnp.where(upd, gi, mi)
        m, mi = lax.fori_loop(
            0, NCH, amax_body,
            (jnp.full((16,), NEG_INF, jnp.float32), gio_v[pl.ds(0, 16)]),
            unroll=False)
        vmax = jnp.max(m)
        jloc = jnp.min(jnp.where(m == vmax, mi, jnp.float32(1e9)))
        li = jloc.astype(jnp.int32) - base

        g = plsc.load_gather(cat_v, [gidx_plane + li])
        cand = jnp.where(iota <= 4, g,
                         jnp.where(iota == 5, vmax, jloc))
        stage_v[...] = cand
        pltpu.sync_copy(stage_v, cands_sh.at[pl.ds(wid * 16, 16)])
        plsc.subcore_barrier()
        pltpu.sync_copy(cands_sh, allc_v)
        plsc.subcore_barrier()

        vals = plsc.load_gather(allc_v, [lane5])
        gv = jnp.max(vals)
        wr = jnp.min(jnp.where(vals == gv, iota, jnp.int32(9999)))
        wrow = plsc.load_gather(allc_v, [wr * 16 + iota])
        wx1 = wrow[0]
        wy1 = wrow[1]
        wx2 = wrow[2]
        wy2 = wrow[3]
        warea = wrow[4]
        wv = wrow[5]
        wjf = wrow[6]

        def sweep_body(c, _):
            ds = pl.ds(c * 16, 16)
            x1 = jnp.maximum(wx1, cat_v[pl.ds(c * 16, 16)])
            y1 = jnp.maximum(wy1, cat_v[pl.ds(PW + c * 16, 16)])
            x2 = jnp.minimum(wx2, cat_v[pl.ds(2 * PW + c * 16, 16)])
            y2 = jnp.minimum(wy2, cat_v[pl.ds(3 * PW + c * 16, 16)])
            inter = jnp.maximum(x2 - x1, 0.0) * jnp.maximum(y2 - y1, 0.0)
            area2 = cat_v[pl.ds(4 * PW + c * 16, 16)]
            union = warea + area2 - inter
            iou = inter / jnp.maximum(union, 1e-9)
            sup = (iou > NMS_THRESH) | (gio_v[ds] == wjf)
            work_v[ds] = jnp.where(sup, NEG_INF, work_v[ds])
            return 0
        lax.fori_loop(0, NCH, sweep_body, 0, unroll=False)

        row = jnp.where(iota < 4, wrow, jnp.where(iota == 4, wv, 0.0))
        row = jnp.where(wv > NEG_INF, row, jnp.zeros((16,), jnp.float32))
        out_v[pl.ds(it * 16, 16)] = row
        return 0

    lax.fori_loop(0, MAX_DET, round_body, 0, unroll=False)

    @pl.when(wid == 0)
    def _():
        pltpu.sync_copy(out_v, out_hbm)


@jax.jit
def _nms(cat, sc):
    mesh = plsc.VectorSubcoreMesh(core_axis_name="c", subcore_axis_name="s",
                                  num_cores=1)
    f = pl.kernel(
        _nms_body,
        out_type=jax.ShapeDtypeStruct((MAX_DET * 16,), jnp.float32),
        mesh=mesh,
        compiler_params=pltpu.CompilerParams(needs_layout_passes=False),
        scratch_types=[
            pltpu.VMEM((5 * PW,), jnp.float32),
            pltpu.VMEM((PW,), jnp.float32),
            pltpu.VMEM((PW,), jnp.float32),
            pltpu.VMEM((16,), jnp.float32),
            pltpu.VMEM((NW * 16,), jnp.float32),
            pltpu.VMEM((MAX_DET * 16,), jnp.float32),
            pltpu.VMEM_SHARED((NW * 16,), jnp.float32),
        ],
    )
    return f(cat, sc)


def kernel(boxes, scores):
    b = jnp.pad(boxes, ((0, NP - N), (0, 0)))
    s = jnp.pad(scores, ((0, NP - N),), constant_values=-1.0)
    cat = b.T.reshape(-1)
    out = _nms(cat, s)
    return out.reshape(MAX_DET, 16)[:, :5]


# speedup vs baseline: 20.1428x; 3.7055x over previous
import functools

import jax
import jax.numpy as jnp
from jax import lax
from jax.experimental import pallas as pl
from jax.experimental.pallas import tpu as pltpu
from jax.experimental.pallas import tpu_sc as plsc

N = 20000
NP = 20480
NW = 16
PW = NP // NW
NCH = PW // 16
NSEG = NCH
KPAD = 112
MAX_DET = 100
SCORE_THRESH = 0.05
NMS_THRESH = 0.5
NEG_INF = float("-inf")


def _nms_body(cat_hbm, sc_hbm, out_hbm, cat_v, work_v, seg_v, kcat_v,
              stage_v, allc_v, out_v, cands_sh):
    wid = lax.axis_index("s")
    iota = lax.broadcasted_iota(jnp.int32, (16,), 0)
    iota_f = iota.astype(jnp.float32)
    lane0 = iota == 0
    lane5 = iota * 16 + 5
    gidx_plane = jnp.minimum(iota, 4) * PW
    neg16 = jnp.full((16,), NEG_INF, jnp.float32)
    base = wid * PW

    for k in range(4):
        pltpu.sync_copy(cat_hbm.at[pl.ds(k * NP + base, PW)],
                        cat_v.at[pl.ds(k * PW, PW)])
    pltpu.sync_copy(sc_hbm.at[pl.ds(base, PW)], work_v)

    def init_body(c, _):
        ds = pl.ds(c * 16, 16)
        x1 = cat_v[pl.ds(c * 16, 16)]
        y1 = cat_v[pl.ds(PW + c * 16, 16)]
        x2 = cat_v[pl.ds(2 * PW + c * 16, 16)]
        y2 = cat_v[pl.ds(3 * PW + c * 16, 16)]
        area = jnp.maximum(x2 - x1, 0.0) * jnp.maximum(y2 - y1, 0.0)
        cat_v[pl.ds(4 * PW + c * 16, 16)] = area
        s = work_v[ds]
        w = jnp.where(s >= SCORE_THRESH, s, NEG_INF)
        work_v[ds] = w
        plsc.store_scatter(seg_v, [jnp.full((16,), c, jnp.int32)],
                           jnp.full((16,), jnp.max(w), jnp.float32),
                           mask=lane0)
        return 0
    lax.fori_loop(0, NCH, init_body, 0, unroll=False)

    for p, val in enumerate((2.0, 2.0, -2.0, -2.0, 0.0)):
        for c in range(KPAD // 16):
            kcat_v[pl.ds(p * KPAD + c * 16, 16)] = jnp.full(
                (16,), val, jnp.float32)

    def local_pick():
        def seg_amax(c, carry):
            m, mi = carry
            v = seg_v[pl.ds(c * 16, 16)]
            gi = c * 16.0 + iota_f
            upd = v > m
            return jnp.where(upd, v, m), jnp.where(upd, gi, mi)
        m, mi = lax.fori_loop(0, NSEG // 16, seg_amax, (neg16, iota_f),
                              unroll=True)
        vmax = jnp.max(m)
        sbest = jnp.min(jnp.where(m == vmax, mi, jnp.float32(1e9)))
        sbest = sbest.astype(jnp.int32)
        wk = work_v[pl.ds(sbest * 16, 16)]
        lane = jnp.min(jnp.where(wk == vmax, iota, jnp.int32(9999)))
        return vmax, sbest * 16 + lane

    def kept_test(it, vmax, li):
        g = plsc.load_gather(cat_v, [gidx_plane + li])
        cx1, cy1, cx2, cy2, carea = g[0], g[1], g[2], g[3], g[4]
        nch = (it + 15) // 16

        def tb(k, acc):
            kx1 = kcat_v[pl.ds(k * 16, 16)]
            ky1 = kcat_v[pl.ds(KPAD + k * 16, 16)]
            kx2 = kcat_v[pl.ds(2 * KPAD + k * 16, 16)]
            ky2 = kcat_v[pl.ds(3 * KPAD + k * 16, 16)]
            karea = kcat_v[pl.ds(4 * KPAD + k * 16, 16)]
            x1 = jnp.maximum(kx1, cx1)
            y1 = jnp.maximum(ky1, cy1)
            x2 = jnp.minimum(kx2, cx2)
            y2 = jnp.minimum(ky2, cy2)
            inter = jnp.maximum(x2 - x1, 0.0) * jnp.maximum(y2 - y1, 0.0)
            union = karea + carea - inter
            iou = inter / jnp.maximum(union, 1e-9)
            return acc | (iou > NMS_THRESH)
        acc = lax.fori_loop(0, nch, tb, iota < 0, unroll=False)
        return jnp.any(acc) & (vmax > NEG_INF)

    def mark(li):
        plsc.store_scatter(work_v, [jnp.full((16,), li, jnp.int32)],
                           neg16, mask=lane0)
        s = li // 16
        newmax = jnp.max(work_v[pl.ds(s * 16, 16)])
        plsc.store_scatter(seg_v, [jnp.full((16,), s, jnp.int32)],
                           jnp.full((16,), newmax, jnp.float32), mask=lane0)

    def round_body(it, _):
        vmax0, li0 = local_pick()
        fail0 = kept_test(it, vmax0, li0)

        def cond(carry):
            return carry[2]

        def body(carry):
            _, li, _ = carry
            mark(li)
            vmax2, li2 = local_pick()
            return vmax2, li2, kept_test(it, vmax2, li2)
        vmax, li, _ = lax.while_loop(cond, body, (vmax0, li0, fail0))

        g = plsc.load_gather(cat_v, [gidx_plane + li])
        cand = jnp.where(iota <= 4, g, vmax)
        stage_v[...] = cand
        par = (it & 1) * (NW * 16)
        pltpu.sync_copy(stage_v, cands_sh.at[pl.ds(par + wid * 16, 16)])
        plsc.subcore_barrier()
        pltpu.sync_copy(cands_sh.at[pl.ds(par, NW * 16)], allc_v)

        vals = plsc.load_gather(allc_v, [lane5])
        gv = jnp.max(vals)
        wr = jnp.min(jnp.where(vals == gv, iota, jnp.int32(9999)))
        wrow = plsc.load_gather(allc_v, [wr * 16 + iota])
        wv = wrow[5]
        valid = wv > NEG_INF

        @pl.when(wr == wid)
        def _():
            mark(li)

        sent = jnp.where(iota < 2, 2.0, jnp.where(iota < 4, -2.0, 0.0))
        app = jnp.where(valid, wrow, sent)
        for p in range(5):
            plsc.store_scatter(
                kcat_v, [jnp.full((16,), p * KPAD, jnp.int32) + it],
                jnp.full((16,), app[p], jnp.float32), mask=lane0)

        row = jnp.where(iota < 4, wrow, jnp.where(iota == 4, wv, 0.0))
        row = jnp.where(valid, row, jnp.zeros((16,), jnp.float32))
        out_v[pl.ds(it * 16, 16)] = row
        return 0

    lax.fori_loop(0, MAX_DET, round_body, 0, unroll=False)

    @pl.when(wid == 0)
    def _():
        pltpu.sync_copy(out_v, out_hbm)


@jax.jit
def _nms(cat, sc):
    mesh = plsc.VectorSubcoreMesh(core_axis_name="c", subcore_axis_name="s",
                                  num_cores=1)
    f = pl.kernel(
        _nms_body,
        out_type=jax.ShapeDtypeStruct((MAX_DET * 16,), jnp.float32),
        mesh=mesh,
        compiler_params=pltpu.CompilerParams(needs_layout_passes=False),
        scratch_types=[
            pltpu.VMEM((5 * PW,), jnp.float32),
            pltpu.VMEM((PW,), jnp.float32),
            pltpu.VMEM((NSEG,), jnp.float32),
            pltpu.VMEM((5 * KPAD,), jnp.float32),
            pltpu.VMEM((16,), jnp.float32),
            pltpu.VMEM((NW * 16,), jnp.float32),
            pltpu.VMEM((MAX_DET * 16,), jnp.float32),
            pltpu.VMEM_SHARED((2 * NW * 16,), jnp.float32),
        ],
    )
    return f(cat, sc)


def kernel(boxes, scores):
    b = jnp.pad(boxes, ((0, NP - N), (0, 0)))
    s = jnp.pad(scores, ((0, NP - N),), constant_values=-1.0)
    cat = b.T.reshape(-1)
    out = _nms(cat, s)
    return out.reshape(MAX_DET, 16)[:, :5]


# speedup vs baseline: 26.3535x; 1.3083x over previous
import functools

import jax
import jax.numpy as jnp
from jax import lax
from jax.experimental import pallas as pl
from jax.experimental.pallas import tpu as pltpu
from jax.experimental.pallas import tpu_sc as plsc

N = 20000
NP = 20480
NS1 = NP // 16
NS2 = NS1 // 16
KPAD = 112
MAX_DET = 100
SCORE_THRESH = 0.05
NMS_THRESH = 0.5
NEG_INF = float("-inf")


def _nms_body(cat_hbm, sc_hbm, out_hbm, cat_v, work_v, seg1_v, seg2_v,
              kcat_v, out_v, sem):
    wid = lax.axis_index("s")

    @pl.when(wid == 0)
    def _():
        iota = lax.broadcasted_iota(jnp.int32, (16,), 0)
        iota_f = iota.astype(jnp.float32)
        iota16 = iota * 16
        lane0 = iota == 0
        gplane = jnp.minimum(iota, 3) * NP
        neg16 = jnp.full((16,), NEG_INF, jnp.float32)

        cdma = pltpu.async_copy(cat_hbm, cat_v, sem)
        pltpu.sync_copy(sc_hbm, work_v)

        def s1(c, _):
            m = neg16
            for j in range(16):
                s = plsc.load_gather(work_v, [c * 256 + iota16 + j])
                m = jnp.maximum(m, jnp.where(s >= SCORE_THRESH, s, NEG_INF))
            seg1_v[pl.ds(c * 16, 16)] = m
            return 0
        lax.fori_loop(0, NS2, s1, 0, unroll=False)

        def s2(c, _):
            m = neg16
            for j in range(16):
                m = jnp.maximum(m, plsc.load_gather(seg1_v,
                                                    [c * 256 + iota16 + j]))
            seg2_v[pl.ds(c * 16, 16)] = m
            return 0
        lax.fori_loop(0, NS2 // 16, s2, 0, unroll=True)

        for p, val in enumerate((2.0, 2.0, -2.0, -2.0, 0.0)):
            for c in range(KPAD // 16):
                kcat_v[pl.ds(p * KPAD + c * 16, 16)] = jnp.full(
                    (16,), val, jnp.float32)

        cdma.wait()

        def pick():
            def seg_scan(c, carry):
                m, mi = carry
                v = seg2_v[pl.ds(c * 16, 16)]
                upd = v > m
                return jnp.where(upd, v, m), jnp.where(upd, c * 16.0 + iota_f,
                                                       mi)
            m, mi = lax.fori_loop(0, NS2 // 16, seg_scan, (neg16, iota_f),
                                  unroll=True)
            vmax = jnp.max(m)
            j2 = jnp.min(jnp.where(m == vmax, mi, jnp.float32(1e9)))
            j2 = j2.astype(jnp.int32)
            c1 = seg1_v[pl.ds(j2 * 16, 16)]
            j1 = j2 * 16 + jnp.minimum(plsc.all_reduce_ffs(c1 == vmax)[0], 15)
            c0 = work_v[pl.ds(j1 * 16, 16)]
            g = j1 * 16 + jnp.minimum(plsc.all_reduce_ffs(c0 == vmax)[0], 15)
            return vmax, g

        def kept_test(it, vmax, g):
            gc = plsc.load_gather(cat_v, [gplane + g])
            cx1, cy1, cx2, cy2 = gc[0], gc[1], gc[2], gc[3]
            carea = jnp.maximum(cx2 - cx1, 0.0) * jnp.maximum(cy2 - cy1, 0.0)
            nch = (it + 15) // 16

            def tb(k, acc):
                kx1 = kcat_v[pl.ds(k * 16, 16)]
                ky1 = kcat_v[pl.ds(KPAD + k * 16, 16)]
                kx2 = kcat_v[pl.ds(2 * KPAD + k * 16, 16)]
                ky2 = kcat_v[pl.ds(3 * KPAD + k * 16, 16)]
                karea = kcat_v[pl.ds(4 * KPAD + k * 16, 16)]
                x1 = jnp.maximum(kx1, cx1)
                y1 = jnp.maximum(ky1, cy1)
                x2 = jnp.minimum(kx2, cx2)
                y2 = jnp.minimum(ky2, cy2)
                inter = jnp.maximum(x2 - x1, 0.0) * jnp.maximum(y2 - y1, 0.0)
                union = karea + carea - inter
                iou = inter / jnp.maximum(union, 1e-9)
                return acc | (iou > NMS_THRESH)
            acc = lax.fori_loop(0, nch, tb, iota < 0, unroll=False)
            return jnp.any(acc) & (vmax > NEG_INF)

        def mark(g):
            plsc.store_scatter(work_v, [jnp.full((16,), g, jnp.int32)],
                               neg16, mask=lane0)
            j1 = g // 16
            c0 = work_v[pl.ds(j1 * 16, 16)]
            nm1 = jnp.max(jnp.where(c0 >= SCORE_THRESH, c0, NEG_INF))
            plsc.store_scatter(seg1_v, [jnp.full((16,), j1, jnp.int32)],
                               jnp.full((16,), nm1, jnp.float32), mask=lane0)
            j2 = j1 // 16
            nm2 = jnp.max(seg1_v[pl.ds(j2 * 16, 16)])
            plsc.store_scatter(seg2_v, [jnp.full((16,), j2, jnp.int32)],
                               jnp.full((16,), nm2, jnp.float32), mask=lane0)

        def round_body(it, _):
            vmax0, g0 = pick()
            fail0 = kept_test(it, vmax0, g0)

            def body(carry):
                _, g, _ = carry
                mark(g)
                vmax2, g2 = pick()
                return vmax2, g2, kept_test(it, vmax2, g2)
            vmax, g, _ = lax.while_loop(lambda c: c[2], body,
                                        (vmax0, g0, fail0))
            mark(g)
            valid = vmax > NEG_INF

            gc = plsc.load_gather(cat_v, [gplane + g])
            sent = jnp.where(iota < 2, 2.0, jnp.where(iota < 4, -2.0, 0.0))
            app = jnp.where(valid, gc, sent)
            ax1, ay1, ax2, ay2 = app[0], app[1], app[2], app[3]
            aarea = jnp.maximum(ax2 - ax1, 0.0) * jnp.maximum(ay2 - ay1, 0.0)
            for p, v in enumerate((ax1, ay1, ax2, ay2, aarea)):
                plsc.store_scatter(
                    kcat_v, [jnp.full((16,), p * KPAD, jnp.int32) + it],
                    jnp.full((16,), v, jnp.float32), mask=lane0)

            row = jnp.where(iota < 4, gc, jnp.where(iota == 4, vmax, 0.0))
            row = jnp.where(valid, row, jnp.zeros((16,), jnp.float32))
            out_v[pl.ds(it * 16, 16)] = row
            return 0

        lax.fori_loop(0, MAX_DET, round_body, 0, unroll=False)
        pltpu.sync_copy(out_v, out_hbm)


@jax.jit
def _nms(cat, sc):
    mesh = plsc.VectorSubcoreMesh(core_axis_name="c", subcore_axis_name="s",
                                  num_cores=1)
    f = pl.kernel(
        _nms_body,
        out_type=jax.ShapeDtypeStruct((MAX_DET * 16,), jnp.float32),
        mesh=mesh,
        compiler_params=pltpu.CompilerParams(needs_layout_passes=False),
        scratch_types=[
            pltpu.VMEM((4 * NP,), jnp.float32),
            pltpu.VMEM((NP,), jnp.float32),
            pltpu.VMEM((NS1,), jnp.float32),
            pltpu.VMEM((NS2,), jnp.float32),
            pltpu.VMEM((5 * KPAD,), jnp.float32),
            pltpu.VMEM((MAX_DET * 16,), jnp.float32),
            pltpu.SemaphoreType.DMA,
        ],
    )
    return f(cat, sc)


def kernel(boxes, scores):
    b = jnp.pad(boxes, ((0, NP - N), (0, 0)))
    s = jnp.pad(scores, ((0, NP - N),), constant_values=-1.0)
    cat = b.T.reshape(-1)
    out = _nms(cat, s)
    return out.reshape(MAX_DET, 16)[:, :5]
